# baseline (speedup 1.0000x reference)
import jax
import jax.numpy as jnp
from jax.experimental import pallas as pl
from jax.experimental.pallas import tpu as pltpu

EOS_ID = 2
HARD_THRED = 1.0
LOSS_WEIGHT = 1.0
B, L, V = 32, 64, 32000


def _fused_kernel(g_ref, ul_ref, ut_ref, trow_ref, tcol_ref, tok_ref, out_ref,
                  prow_ref, pcol_ref, ilrow_ref, ilcol_ref):
    i = pl.program_id(0)
    x = g_ref[...]
    m = jnp.max(x, axis=-1, keepdims=True)
    s = jnp.sum(jnp.exp(x - m), axis=-1, keepdims=True)
    col = -jnp.log(s)

    iota = jax.lax.broadcasted_iota(jnp.int32, (L, 1), 0)
    tok = tok_ref[0]
    is_eos = tok == EOS_ID
    first_eos = jnp.min(jnp.where(is_eos, iota, L))
    maskb = iota <= first_eos
    maskf = maskb.astype(jnp.float32)
    mask_sum = jnp.sum(maskf)

    col = jnp.where(maskb, col, -jnp.inf)
    col = jnp.where(col == 0.0, -jnp.inf, col)
    logits = ul_ref[0] * maskf

    acc = jnp.float32(0.0)
    avail = iota < L
    for _ in range(5):
        val = jnp.where(avail, col, -jnp.inf)
        mx = jnp.max(val)
        fi = jnp.min(jnp.where((val == mx) & avail, iota, L))
        pick = iota == fi
        acc = acc + jnp.sum(jnp.where(pick, logits, 0.0))
        avail = avail & jnp.logical_not(pick)
    preds_i = acc / mask_sum
    il_i = jnp.sum(ut_ref[0] * maskf) / L

    lane = jax.lax.broadcasted_iota(jnp.int32, (1, B), 1)
    sub = jax.lax.broadcasted_iota(jnp.int32, (B, 1), 0)
    prow_ref[...] = jnp.where(lane == i, preds_i, prow_ref[...])
    pcol_ref[...] = jnp.where(sub == i, preds_i, pcol_ref[...])
    ilrow_ref[...] = jnp.where(lane == i, il_i, ilrow_ref[...])
    ilcol_ref[...] = jnp.where(sub == i, il_i, ilcol_ref[...])

    @pl.when(i == B - 1)
    def _():
        p_row = prow_ref[...]
        p_col = pcol_ref[...]
        il_row = ilrow_ref[...]
        il_col = ilcol_ref[...]
        dt = jnp.abs(trow_ref[...] - tcol_ref[...])
        masks_time = ((dt < 0.12) & (dt > 0.0)).astype(jnp.float32)
        dlab = il_row - il_col
        masks = jnp.sign(dlab) * masks_time
        adl = jnp.abs(dlab)
        masks_hard = ((adl < HARD_THRED) & (adl > 0.0)).astype(jnp.float32) * masks_time
        rank_loss = masks_hard * jnp.maximum(-masks * (p_row - p_col), 0.0)
        loss = jnp.sum(rank_loss) / (jnp.sum(masks_hard) + 1e-08)
        out_ref[...] = jnp.reshape(loss * LOSS_WEIGHT, (1, 1))


def kernel(u_logits, u_target_ids, g_logits, times, u_tokens_ids):
    loss = pl.pallas_call(
        _fused_kernel,
        grid=(B,),
        in_specs=[
            pl.BlockSpec((L, V), lambda i: (i, 0)),
            pl.BlockSpec((1, L, 1), lambda i: (i, 0, 0)),
            pl.BlockSpec((1, L, 1), lambda i: (i, 0, 0)),
            pl.BlockSpec((1, B), lambda i: (0, 0)),
            pl.BlockSpec((B, 1), lambda i: (0, 0)),
            pl.BlockSpec((1, L, 1), lambda i: (i, 0, 0)),
        ],
        out_specs=pl.BlockSpec((1, 1), lambda i: (0, 0)),
        out_shape=jax.ShapeDtypeStruct((1, 1), jnp.float32),
        scratch_shapes=[
            pltpu.VMEM((1, B), jnp.float32),
            pltpu.VMEM((B, 1), jnp.float32),
            pltpu.VMEM((1, B), jnp.float32),
            pltpu.VMEM((B, 1), jnp.float32),
        ],
    )(
        g_logits.reshape(B * L, V),
        u_logits,
        u_target_ids,
        times.reshape(1, B),
        times.reshape(B, 1),
        u_tokens_ids.reshape(B, L, 1),
    )
    return loss.reshape(())
